# baseline (speedup 1.0000x reference)
import functools

import jax
import jax.numpy as jnp
from jax import lax
from jax.experimental import pallas as pl
from jax.experimental.pallas import tpu as pltpu
from jax.experimental.pallas import tpu_sc as plsc

_B = 4096
_T = 200
_EMB = 32
_V4 = 250000

_NC = 2
_NS = 16
_NW = _NC * _NS
_BW = _B // _NW
_CT = 2
_CN = _CT * _BW
_NCHUNK = _T // _CT
_L = 16
_K = _BW // _L
_OBR = _CT * _EMB
_PITCH = _BW + 1

_mesh = plsc.VectorSubcoreMesh(core_axis_name="c", subcore_axis_name="s")


@functools.partial(
    pl.kernel,
    mesh=_mesh,
    out_type=jax.ShapeDtypeStruct((_T * _EMB, _B), jnp.float32),
    scratch_types=[
        pltpu.VMEM((_T, _BW), jnp.int32),
        pltpu.VMEM((_CN,), jnp.int32),
        pltpu.VMEM((_CN,), jnp.int32),
        pltpu.VMEM((_CN,), jnp.int32),
        pltpu.VMEM((_CN,), jnp.int32),
        pltpu.VMEM((_CN, 128), jnp.float32),
        pltpu.VMEM((_CN, 128), jnp.float32),
        pltpu.VMEM((_OBR, _PITCH), jnp.float32),
        pltpu.VMEM((_OBR, _PITCH), jnp.float32),
        pltpu.SemaphoreType.DMA,
        pltpu.SemaphoreType.DMA,
        pltpu.SemaphoreType.DMA,
        pltpu.SemaphoreType.DMA,
    ],
    compiler_params=pltpu.CompilerParams(
        needs_layout_passes=False, disable_bounds_checks=True),
)
def _sc_gather(x_hbm, table_hbm, out_hbm, xi, ig0, ig1, ir0, ir1,
               rows0, rows1, ob0, ob1,
               sg0, sg1, sw0, sw1):
    wid = lax.axis_index("s") * _NC + lax.axis_index("c")
    b0 = wid * _BW
    ig = (ig0, ig1)
    ir = (ir0, ir1)
    rows = (rows0, rows1)
    ob = (ob0, ob1)
    sg = (sg0, sg1)
    sw = (sw0, sw1)

    iota = lax.iota(jnp.int32, _L)

    pltpu.sync_copy(x_hbm.at[:, pl.ds(b0, _BW)], xi)

    def stage(ci, p):
        @pl.loop(0, _CT)
        def _t(t):
            for k in range(_K):
                v = xi[ci * _CT + t, pl.ds(k * _L, _L)]
                o = t * _BW + k * _L
                ig[p][pl.ds(o, _L)] = lax.shift_right_logical(v, 2)
                ir[p][pl.ds(o, _L)] = lax.bitwise_and(v, 3)

        pltpu.async_copy(table_hbm.at[ig[p]], rows[p], sg[p])

    def drain(ci, p):
        pltpu.make_async_copy(table_hbm.at[ig[p]], rows[p], sg[p]).wait()

        @pl.when(ci >= 2)
        def _wait_wb():
            pltpu.make_async_copy(
                ob[p].at[:, pl.ds(0, _BW)],
                out_hbm.at[pl.ds(0, _OBR), pl.ds(b0, _BW)],
                sw[p]).wait()

        for t in range(_CT):
            row0 = t * _EMB + iota
            row1 = row0 + _L

            @pl.loop(0, _BW, unroll=8)
            def _b(b):
                row = t * _BW + b
                bvec = lax.broadcast(row, (_L,))
                cvec = plsc.load_gather(ir[p], [bvec])
                m_hi = cvec >= 2
                m_od = lax.bitwise_and(cvec, 1) > 0
                r = rows[p].at[row]
                v0 = r[pl.ds(0, _L)]
                v1 = r[pl.ds(_L, _L)]
                v2 = r[pl.ds(2 * _L, _L)]
                v3 = r[pl.ds(3 * _L, _L)]
                v4 = r[pl.ds(4 * _L, _L)]
                v5 = r[pl.ds(5 * _L, _L)]
                v6 = r[pl.ds(6 * _L, _L)]
                v7 = r[pl.ds(7 * _L, _L)]
                w0 = jnp.where(m_hi, jnp.where(m_od, v6, v4),
                               jnp.where(m_od, v2, v0))
                w1 = jnp.where(m_hi, jnp.where(m_od, v7, v5),
                               jnp.where(m_od, v3, v1))
                bv = lax.broadcast(b, (_L,))
                plsc.store_scatter(ob[p], [row0, bv], w0)
                plsc.store_scatter(ob[p], [row1, bv], w1)

        pltpu.async_copy(
            ob[p].at[:, pl.ds(0, _BW)],
            out_hbm.at[pl.ds(ci * _OBR, _OBR), pl.ds(b0, _BW)],
            sw[p])

    stage(0, 0)
    stage(1, 1)

    @pl.loop(0, _NCHUNK // 2 - 1)
    def body(g):
        for p in range(2):
            i = 2 * g + p
            drain(i, p)
            stage(i + 2, p)

    drain(_NCHUNK - 2, 0)
    drain(_NCHUNK - 1, 1)


def kernel(x, table):
    res = _sc_gather(x.T, table.reshape(_V4, 128))
    return res.reshape(_T, _EMB, _B).transpose(2, 0, 1)


# speedup vs baseline: 1.1415x; 1.1415x over previous
import functools

import jax
import jax.numpy as jnp
from jax import lax
from jax.experimental import pallas as pl
from jax.experimental.pallas import tpu as pltpu
from jax.experimental.pallas import tpu_sc as plsc

_B = 4096
_T = 200
_EMB = 32
_N = _B * _T

_NC = 2
_NS = 16
_NW = _NC * _NS
_PER_W = _N // _NW
_CHUNK = 1280
_NCHUNK = _PER_W // _CHUNK

_mesh = plsc.VectorSubcoreMesh(core_axis_name="c", subcore_axis_name="s")


@functools.partial(
    pl.kernel,
    mesh=_mesh,
    out_type=jax.ShapeDtypeStruct((_N, _EMB), jnp.float32),
    scratch_types=[
        pltpu.VMEM((_NCHUNK, _CHUNK), jnp.int32),
        pltpu.VMEM((_CHUNK, _EMB), jnp.float32),
        pltpu.VMEM((_CHUNK, _EMB), jnp.float32),
        pltpu.SemaphoreType.DMA,
        pltpu.SemaphoreType.DMA,
        pltpu.SemaphoreType.DMA,
        pltpu.SemaphoreType.DMA,
    ],
    compiler_params=pltpu.CompilerParams(use_tc_tiling_on_sc=False),
)
def _sc_gather(idx_hbm, table_hbm, out_hbm, idx_v, rows0, rows1,
               sg0, sg1, sw0, sw1):
    wid = lax.axis_index("s") * _NC + lax.axis_index("c")
    base = wid * _PER_W
    rows = (rows0, rows1)
    sg = (sg0, sg1)
    sw = (sw0, sw1)

    pltpu.sync_copy(idx_hbm.at[wid], idx_v)

    @pl.loop(0, _NCHUNK // 2)
    def body(g):
        for b in range(2):
            i = 2 * g + b

            @pl.when(g > 0)
            def _wait_wb():
                pltpu.make_async_copy(
                    rows[b], out_hbm.at[pl.ds(base, _CHUNK)], sw[b]).wait()

            pltpu.async_copy(table_hbm.at[idx_v.at[i]], rows[b], sg[b])

        for b in range(2):
            i = 2 * g + b
            pltpu.make_async_copy(
                table_hbm.at[idx_v.at[i]], rows[b], sg[b]).wait()
            pltpu.async_copy(
                rows[b], out_hbm.at[pl.ds(base + i * _CHUNK, _CHUNK)], sw[b])

    for b in range(2):
        pltpu.make_async_copy(
            rows[b], out_hbm.at[pl.ds(base, _CHUNK)], sw[b]).wait()


def kernel(x, table):
    flat = _sc_gather(x.reshape(_NW, _NCHUNK, _CHUNK), table)
    return flat.reshape(_B, _T, _EMB)


# speedup vs baseline: 1.1443x; 1.0025x over previous
import functools

import jax
import jax.numpy as jnp
from jax import lax
from jax.experimental import pallas as pl
from jax.experimental.pallas import tpu as pltpu
from jax.experimental.pallas import tpu_sc as plsc

_B = 4096
_T = 200
_EMB = 32
_N = _B * _T
_SPLIT = 2
_NH = _N // _SPLIT

_NC = 2
_NS = 16
_NW = _NC * _NS
_PER_W = _NH // _NW
_CHUNK = 1280
_NCHUNK = _PER_W // _CHUNK

_mesh = plsc.VectorSubcoreMesh(core_axis_name="c", subcore_axis_name="s")


@functools.partial(
    pl.kernel,
    mesh=_mesh,
    out_type=jax.ShapeDtypeStruct((_NH, _EMB), jnp.float32),
    scratch_types=[
        pltpu.VMEM((_NCHUNK, _CHUNK), jnp.int32),
        pltpu.VMEM((_CHUNK, _EMB), jnp.float32),
        pltpu.VMEM((_CHUNK, _EMB), jnp.float32),
        pltpu.SemaphoreType.DMA,
        pltpu.SemaphoreType.DMA,
        pltpu.SemaphoreType.DMA,
        pltpu.SemaphoreType.DMA,
    ],
    compiler_params=pltpu.CompilerParams(use_tc_tiling_on_sc=False),
)
def _sc_gather(idx_hbm, table_hbm, out_hbm, idx_v, rows0, rows1,
               sg0, sg1, sw0, sw1):
    wid = lax.axis_index("s") * _NC + lax.axis_index("c")
    base = wid * _PER_W
    rows = (rows0, rows1)
    sg = (sg0, sg1)
    sw = (sw0, sw1)

    pltpu.sync_copy(idx_hbm.at[wid], idx_v)

    @pl.loop(0, _NCHUNK // 2)
    def body(g):
        for b in range(2):
            i = 2 * g + b

            @pl.when(g > 0)
            def _wait_wb():
                pltpu.make_async_copy(
                    rows[b], out_hbm.at[pl.ds(base, _CHUNK)], sw[b]).wait()

            pltpu.async_copy(table_hbm.at[idx_v.at[i]], rows[b], sg[b])

        for b in range(2):
            i = 2 * g + b
            pltpu.make_async_copy(
                table_hbm.at[idx_v.at[i]], rows[b], sg[b]).wait()
            pltpu.async_copy(
                rows[b], out_hbm.at[pl.ds(base + i * _CHUNK, _CHUNK)], sw[b])

    for b in range(2):
        pltpu.make_async_copy(
            rows[b], out_hbm.at[pl.ds(base, _CHUNK)], sw[b]).wait()


def kernel(x, table):
    xf = x.reshape(_N)
    halves = []
    for s in range(_SPLIT):
        xs = xf[s * _NH:(s + 1) * _NH]
        flat = _sc_gather(xs.reshape(_NW, _NCHUNK, _CHUNK), table)
        halves.append(flat.reshape(_B // _SPLIT, _T, _EMB))
    return jnp.concatenate(halves, axis=0)


# speedup vs baseline: 1.1875x; 1.0377x over previous
import functools

import jax
import jax.numpy as jnp
from jax import lax
from jax.experimental import pallas as pl
from jax.experimental.pallas import tpu as pltpu
from jax.experimental.pallas import tpu_sc as plsc

_B = 4096
_T = 200
_EMB = 32
_N = _B * _T
_SPLIT = 4
_NH = _N // _SPLIT

_NC = 2
_NS = 16
_NW = _NC * _NS
_PER_W = _NH // _NW
_CHUNK = 1600
_NCHUNK = _PER_W // _CHUNK

_mesh = plsc.VectorSubcoreMesh(core_axis_name="c", subcore_axis_name="s")


@functools.partial(
    pl.kernel,
    mesh=_mesh,
    out_type=jax.ShapeDtypeStruct((_NH, _EMB), jnp.float32),
    scratch_types=[
        pltpu.VMEM((_NCHUNK, _CHUNK), jnp.int32),
        pltpu.VMEM((_CHUNK, _EMB), jnp.float32),
        pltpu.VMEM((_CHUNK, _EMB), jnp.float32),
        pltpu.SemaphoreType.DMA,
        pltpu.SemaphoreType.DMA,
        pltpu.SemaphoreType.DMA,
        pltpu.SemaphoreType.DMA,
    ],
    compiler_params=pltpu.CompilerParams(use_tc_tiling_on_sc=False),
)
def _sc_gather(idx_hbm, table_hbm, out_hbm, idx_v, rows0, rows1,
               sg0, sg1, sw0, sw1):
    wid = lax.axis_index("s") * _NC + lax.axis_index("c")
    base = wid * _PER_W
    rows = (rows0, rows1)
    sg = (sg0, sg1)
    sw = (sw0, sw1)

    pltpu.sync_copy(idx_hbm.at[wid], idx_v)

    @pl.loop(0, _NCHUNK // 2)
    def body(g):
        for b in range(2):
            i = 2 * g + b

            @pl.when(g > 0)
            def _wait_wb():
                pltpu.make_async_copy(
                    rows[b], out_hbm.at[pl.ds(base, _CHUNK)], sw[b]).wait()

            pltpu.async_copy(table_hbm.at[idx_v.at[i]], rows[b], sg[b])

        for b in range(2):
            i = 2 * g + b
            pltpu.make_async_copy(
                table_hbm.at[idx_v.at[i]], rows[b], sg[b]).wait()
            pltpu.async_copy(
                rows[b], out_hbm.at[pl.ds(base + i * _CHUNK, _CHUNK)], sw[b])

    for b in range(2):
        pltpu.make_async_copy(
            rows[b], out_hbm.at[pl.ds(base, _CHUNK)], sw[b]).wait()


def kernel(x, table):
    xf = x.reshape(_N)
    halves = []
    for s in range(_SPLIT):
        xs = xf[s * _NH:(s + 1) * _NH]
        flat = _sc_gather(xs.reshape(_NW, _NCHUNK, _CHUNK), table)
        halves.append(flat.reshape(_B // _SPLIT, _T, _EMB))
    return jnp.concatenate(halves, axis=0)
